# baseline (speedup 1.0000x reference)
import math

import jax
import jax.numpy as jnp
from jax.experimental import pallas as pl
from jax.experimental.pallas import tpu as pltpu

_B, _N, _D, _E, _H = 64, 256, 256, 256, 8
_DH = _E // _H
_NEG = -1e9
_SCALE = 1.0 / math.sqrt(_DH)
_G = 4
_TQ = 64
_NT = _N // _TQ


def _critic_body(perm_ref, act_ref, obs0, obs1, obs2, obs3, wd_ref,
                 wlast_ref, bin_ref, wqkv_ref, bqkv_ref, wo_ref, bo_ref,
                 wout_ref, bout_ref, wv_ref, bv_ref, out_ref, h_s, qkv_s):
    b = pl.program_id(0)
    obs_refs = (obs0, obs1, obs2, obs3)
    a = [act_ref[b * _G + g] for g in range(_G)]
    nt = (a[0] + _TQ - 1) // _TQ

    lane = jax.lax.broadcasted_iota(jnp.int32, (1, _N), 1)
    admask = [jnp.where(lane < a[g], 0.0, _NEG) for g in range(_G)]
    rb = [a[g].astype(jnp.float32) * (1.0 / _N) * wlast_ref[...]
          + bin_ref[...] for g in range(_G)]

    def p1(t, c):
        r0 = pl.multiple_of(t * _TQ, _TQ)
        for g in range(_G):
            obs_t = obs_refs[g][0, pl.ds(r0, _TQ), :]
            h_t = jnp.dot(obs_t, wd_ref[...],
                          preferred_element_type=jnp.float32) + rb[g]
            h_t = jnp.maximum(h_t, 0.0)
            qkv_t = jnp.dot(h_t, wqkv_ref[...],
                            preferred_element_type=jnp.float32) + bqkv_ref[...]
            h_s[g, pl.ds(r0, _TQ), :] = h_t
            qkv_s[g, pl.ds(r0, _TQ), :] = qkv_t
        return c

    jax.lax.fori_loop(0, nt, p1, 0)

    def pz(t, c):
        r0 = pl.multiple_of(t * _TQ, _TQ)
        for g in range(_G):
            qkv_s[g, pl.ds(r0, _TQ), _E:] = jnp.zeros((_TQ, 2 * _E),
                                                      jnp.float32)
        return c

    jax.lax.fori_loop(nt, _NT, pz, 0)

    def p2(t, accs):
        r0 = pl.multiple_of(t * _TQ, _TQ)
        new = []
        for g in range(_G):
            qt = qkv_s[g, pl.ds(r0, _TQ), 0:_E]
            parts = []
            for i in range(_H):
                qh = qt[:, i * _DH:(i + 1) * _DH]
                kh = qkv_s[g, :, _E + i * _DH:_E + (i + 1) * _DH]
                vh = qkv_s[g, :, 2 * _E + i * _DH:2 * _E + (i + 1) * _DH]
                s = jax.lax.dot_general(qh, kh, (((1,), (1,)), ((), ())),
                                        preferred_element_type=jnp.float32)
                s = s * _SCALE + admask[g]
                m = jnp.max(s, axis=1, keepdims=True)
                e = jnp.exp(s - m)
                r = 1.0 / jnp.sum(e, axis=1, keepdims=True)
                ctx_h = jnp.dot(e, vh, preferred_element_type=jnp.float32)
                parts.append(ctx_h * r)
            ctx = jnp.concatenate(parts, axis=1)
            attn = jnp.dot(ctx, wo_ref[...],
                           preferred_element_type=jnp.float32) + bo_ref[...]
            pre = attn + h_s[g, pl.ds(r0, _TQ), :]
            rsa = jnp.dot(pre, wout_ref[...],
                          preferred_element_type=jnp.float32) + bout_ref[...]
            rsa = jnp.maximum(rsa, 0.0)
            vr = jax.lax.dot_general(wv_ref[...], rsa,
                                     (((1,), (1,)), ((), ())),
                                     preferred_element_type=jnp.float32)
            vr = vr + bv_ref[...]
            vr = jnp.where(vr >= 0, vr, 0.01 * vr)
            tmask = (jax.lax.broadcasted_iota(jnp.int32, (1, _TQ), 1)
                     + r0) < a[g]
            vr = jnp.where(tmask, vr, 0.0)
            new.append(accs[g] + jnp.sum(vr, axis=1, keepdims=True))
        return tuple(new)

    zero = jnp.zeros((1, 1), jnp.float32)
    accs = jax.lax.fori_loop(0, nt, p2, (zero,) * _G)
    for g in range(_G):
        out_ref[g] = accs[g]


def _obs_map(g):
    return lambda b, perm_ref, act_ref: (perm_ref[b * _G + g], 0, 0)


def _fixed(b, *_):
    return (0, 0)


def kernel(encoded_obs, actives, W_in, b_in, W_qkv, b_qkv, W_o, b_o,
           W_out, b_out, W_v, b_v):
    actv = actives.reshape(_B).astype(jnp.int32)
    perm = jnp.argsort(-actv)
    acts_sorted = actv[perm]
    inv = jnp.argsort(perm)
    grid_spec = pltpu.PrefetchScalarGridSpec(
        num_scalar_prefetch=2,
        grid=(_B // _G,),
        in_specs=[
            pl.BlockSpec((1, _N, _D), _obs_map(0)),
            pl.BlockSpec((1, _N, _D), _obs_map(1)),
            pl.BlockSpec((1, _N, _D), _obs_map(2)),
            pl.BlockSpec((1, _N, _D), _obs_map(3)),
            pl.BlockSpec((_D, _E), _fixed),
            pl.BlockSpec((1, _E), _fixed),
            pl.BlockSpec((1, _E), _fixed),
            pl.BlockSpec((_E, 3 * _E), _fixed),
            pl.BlockSpec((1, 3 * _E), _fixed),
            pl.BlockSpec((_E, _E), _fixed),
            pl.BlockSpec((1, _E), _fixed),
            pl.BlockSpec((_E, _E), _fixed),
            pl.BlockSpec((1, _E), _fixed),
            pl.BlockSpec((1, _E), _fixed),
            pl.BlockSpec((1, 1), _fixed),
        ],
        out_specs=pl.BlockSpec((_G, 1, 1), lambda b, *_: (b, 0, 0)),
        scratch_shapes=[
            pltpu.VMEM((_G, _N, _E), jnp.float32),
            pltpu.VMEM((_G, _N, 3 * _E), jnp.float32),
        ],
    )
    out = pl.pallas_call(
        _critic_body,
        grid_spec=grid_spec,
        out_shape=jax.ShapeDtypeStruct((_B, 1, 1), jnp.float32),
        compiler_params=pltpu.CompilerParams(
            dimension_semantics=("parallel",)),
        name="critic_fused_ragged",
    )(perm, acts_sorted, encoded_obs, encoded_obs, encoded_obs, encoded_obs,
      W_in[:, :_D].T, W_in[:, _D].reshape(1, _E), b_in.reshape(1, _E),
      W_qkv.T, b_qkv.reshape(1, 3 * _E), W_o.T, b_o.reshape(1, _E),
      W_out.T, b_out.reshape(1, _E), W_v, b_v.reshape(1, 1))
    return out.reshape(_B)[inv].reshape(_B, 1)


# speedup vs baseline: 1.4466x; 1.4466x over previous
import math

import jax
import jax.numpy as jnp
from jax.experimental import pallas as pl
from jax.experimental.pallas import tpu as pltpu

_B, _N, _D, _E, _H = 64, 256, 256, 256, 8
_DH = _E // _H
_NEG = -1e9
_LOG2E = math.log2(math.e)
_QSCALE = _LOG2E / math.sqrt(_DH)
_G = 4


def _critic_body(act_ref, obs_ref, wd_ref, wlast_ref, bin_ref, wqkv_ref,
                 bqkv_ref, wo_ref, bo_ref, wout_ref, bout_ref, wv_ref,
                 bv_ref, out_ref, h_s, qkv_s, ctx_s):
    b = pl.program_id(0)
    lane = jax.lax.broadcasted_iota(jnp.int32, (1, _N), 1)

    for g in range(_G):
        a = act_ref[b * _G + g]
        kmask = lane < a
        admask = jnp.where(kmask, 0.0, _NEG)

        h = jnp.dot(obs_ref[g], wd_ref[...],
                    preferred_element_type=jnp.float32)
        h = h + (a.astype(jnp.float32) * (1.0 / _N) * wlast_ref[...]
                 + bin_ref[...])
        h = jnp.maximum(h, 0.0)
        h_s[g] = h

        qkv_s[g] = jnp.dot(h, wqkv_ref[...],
                           preferred_element_type=jnp.float32) + bqkv_ref[...]

        for i in range(_H):
            qh = qkv_s[g, :, i * _DH:(i + 1) * _DH]
            kh = qkv_s[g, :, _E + i * _DH:_E + (i + 1) * _DH]
            vh = qkv_s[g, :, 2 * _E + i * _DH:2 * _E + (i + 1) * _DH]
            s = jax.lax.dot_general(qh, kh, (((1,), (1,)), ((), ())),
                                    preferred_element_type=jnp.float32)
            s = s + admask
            m = jnp.max(s, axis=1, keepdims=True)
            e = jnp.exp2(s - m)
            r = 1.0 / jnp.sum(e, axis=1, keepdims=True)
            ctx_h = jnp.dot(e, vh, preferred_element_type=jnp.float32)
            ctx_s[g, :, i * _DH:(i + 1) * _DH] = ctx_h * r

        attn = jnp.dot(ctx_s[g], wo_ref[...],
                       preferred_element_type=jnp.float32) + bo_ref[...]
        rsa = jnp.dot(attn + h_s[g], wout_ref[...],
                      preferred_element_type=jnp.float32) + bout_ref[...]
        rsa = jnp.maximum(rsa, 0.0)

        vrow = jax.lax.dot_general(wv_ref[...], rsa, (((1,), (1,)), ((), ())),
                                   preferred_element_type=jnp.float32)
        vrow = vrow + bv_ref[...]
        vrow = jnp.where(vrow >= 0, vrow, 0.01 * vrow)
        vrow = jnp.where(kmask, vrow, 0.0)
        out_ref[g] = jnp.sum(vrow, axis=1, keepdims=True)


def _fixed(b, *_):
    return (0, 0)


def kernel(encoded_obs, actives, W_in, b_in, W_qkv, b_qkv, W_o, b_o,
           W_out, b_out, W_v, b_v):
    acts = actives.reshape(_B).astype(jnp.int32)
    qkv_scale = jnp.concatenate(
        [jnp.full((_E,), _QSCALE, jnp.float32),
         jnp.ones((2 * _E,), jnp.float32)])
    wqkv_t = W_qkv.T * qkv_scale[None, :]
    bqkv_row = (b_qkv * qkv_scale).reshape(1, 3 * _E)
    grid_spec = pltpu.PrefetchScalarGridSpec(
        num_scalar_prefetch=1,
        grid=(_B // _G,),
        in_specs=[
            pl.BlockSpec((_G, _N, _D), lambda b, *_: (b, 0, 0)),
            pl.BlockSpec((_D, _E), _fixed),
            pl.BlockSpec((1, _E), _fixed),
            pl.BlockSpec((1, _E), _fixed),
            pl.BlockSpec((_E, 3 * _E), _fixed),
            pl.BlockSpec((1, 3 * _E), _fixed),
            pl.BlockSpec((_E, _E), _fixed),
            pl.BlockSpec((1, _E), _fixed),
            pl.BlockSpec((_E, _E), _fixed),
            pl.BlockSpec((1, _E), _fixed),
            pl.BlockSpec((1, _E), _fixed),
            pl.BlockSpec((1, 1), _fixed),
        ],
        out_specs=pl.BlockSpec((_G, 1, 1), lambda b, *_: (b, 0, 0)),
        scratch_shapes=[
            pltpu.VMEM((_G, _N, _E), jnp.float32),
            pltpu.VMEM((_G, _N, 3 * _E), jnp.float32),
            pltpu.VMEM((_G, _N, _E), jnp.float32),
        ],
    )
    out = pl.pallas_call(
        _critic_body,
        grid_spec=grid_spec,
        out_shape=jax.ShapeDtypeStruct((_B, 1, 1), jnp.float32),
        compiler_params=pltpu.CompilerParams(
            dimension_semantics=("parallel",)),
        name="critic_fused",
    )(acts, encoded_obs, W_in[:, :_D].T, W_in[:, _D].reshape(1, _E),
      b_in.reshape(1, _E), wqkv_t, bqkv_row, W_o.T, b_o.reshape(1, _E),
      W_out.T, b_out.reshape(1, _E), W_v, b_v.reshape(1, 1))
    return out.reshape(_B, 1)


# speedup vs baseline: 2.0255x; 1.4002x over previous
import math

import jax
import jax.numpy as jnp
from jax.experimental import pallas as pl
from jax.experimental.pallas import tpu as pltpu

_B, _N, _D, _E, _H = 64, 256, 256, 256, 8
_DH = _E // _H
_NEG = -1e9
_LOG2E = math.log2(math.e)
_QSCALE = _LOG2E / math.sqrt(_DH)
_G = 4


def _critic_body(act_ref, obs_ref, wd_ref, wlast_ref, bin_ref, wqkv_ref,
                 bqkv_ref, wo_ref, bo_ref, wout_ref, bout_ref, wv_ref,
                 bv_ref, out_ref):
    b = pl.program_id(0)
    lane = jax.lax.broadcasted_iota(jnp.int32, (1, _N), 1)

    for g in range(_G):
        a = act_ref[b * _G + g]
        kmask = lane < a
        admask = jnp.where(kmask, 0.0, _NEG)

        h = jnp.dot(obs_ref[g], wd_ref[...],
                    preferred_element_type=jnp.float32)
        h = h + (a.astype(jnp.float32) * (1.0 / _N) * wlast_ref[...]
                 + bin_ref[...])
        h = jnp.maximum(h, 0.0)

        qkv = jnp.dot(h, wqkv_ref[...],
                      preferred_element_type=jnp.float32) + bqkv_ref[...]

        parts = []
        for i in range(_H):
            qh = qkv[:, i * _DH:(i + 1) * _DH]
            kh = qkv[:, _E + i * _DH:_E + (i + 1) * _DH]
            vh = qkv[:, 2 * _E + i * _DH:2 * _E + (i + 1) * _DH]
            s = jax.lax.dot_general(qh, kh, (((1,), (1,)), ((), ())),
                                    preferred_element_type=jnp.float32)
            s = s + admask
            m = jnp.max(s, axis=1, keepdims=True)
            e = jnp.exp2(s - m)
            r = 1.0 / jnp.sum(e, axis=1, keepdims=True)
            ctx_h = jnp.dot(e, vh, preferred_element_type=jnp.float32)
            parts.append(ctx_h * r)
        ctx = jnp.concatenate(parts, axis=1)

        attn = jnp.dot(ctx, wo_ref[...],
                       preferred_element_type=jnp.float32) + bo_ref[...]
        rsa = jnp.dot(attn + h, wout_ref[...],
                      preferred_element_type=jnp.float32) + bout_ref[...]
        rsa = jnp.maximum(rsa, 0.0)

        vrow = jax.lax.dot_general(wv_ref[...], rsa, (((1,), (1,)), ((), ())),
                                   preferred_element_type=jnp.float32)
        vrow = vrow + bv_ref[...]
        vrow = jnp.where(vrow >= 0, vrow, 0.01 * vrow)
        vrow = jnp.where(kmask, vrow, 0.0)
        out_ref[g] = jnp.sum(vrow, axis=1, keepdims=True)


def _fixed(b, *_):
    return (0, 0)


def kernel(encoded_obs, actives, W_in, b_in, W_qkv, b_qkv, W_o, b_o,
           W_out, b_out, W_v, b_v):
    acts = actives.reshape(_B).astype(jnp.int32)
    qkv_scale = jnp.concatenate(
        [jnp.full((_E,), _QSCALE, jnp.float32),
         jnp.ones((2 * _E,), jnp.float32)])
    wqkv_t = W_qkv.T * qkv_scale[None, :]
    bqkv_row = (b_qkv * qkv_scale).reshape(1, 3 * _E)
    grid_spec = pltpu.PrefetchScalarGridSpec(
        num_scalar_prefetch=1,
        grid=(_B // _G,),
        in_specs=[
            pl.BlockSpec((_G, _N, _D), lambda b, *_: (b, 0, 0)),
            pl.BlockSpec((_D, _E), _fixed),
            pl.BlockSpec((1, _E), _fixed),
            pl.BlockSpec((1, _E), _fixed),
            pl.BlockSpec((_E, 3 * _E), _fixed),
            pl.BlockSpec((1, 3 * _E), _fixed),
            pl.BlockSpec((_E, _E), _fixed),
            pl.BlockSpec((1, _E), _fixed),
            pl.BlockSpec((_E, _E), _fixed),
            pl.BlockSpec((1, _E), _fixed),
            pl.BlockSpec((1, _E), _fixed),
            pl.BlockSpec((1, 1), _fixed),
        ],
        out_specs=pl.BlockSpec((_G, 1, 1), lambda b, *_: (b, 0, 0)),
    )
    out = pl.pallas_call(
        _critic_body,
        grid_spec=grid_spec,
        out_shape=jax.ShapeDtypeStruct((_B, 1, 1), jnp.float32),
        compiler_params=pltpu.CompilerParams(
            dimension_semantics=("parallel",)),
        name="critic_fused",
    )(acts, encoded_obs, W_in[:, :_D].T, W_in[:, _D].reshape(1, _E),
      b_in.reshape(1, _E), wqkv_t, bqkv_row, W_o.T, b_o.reshape(1, _E),
      W_out.T, b_out.reshape(1, _E), W_v, b_v.reshape(1, 1))
    return out.reshape(_B, 1)


# speedup vs baseline: 2.1962x; 1.0843x over previous
import math

import jax
import jax.numpy as jnp
from jax.experimental import pallas as pl
from jax.experimental.pallas import tpu as pltpu

_B, _N, _D, _E, _H = 64, 256, 256, 256, 8
_DH = _E // _H
_NEG = -1e9
_LOG2E = math.log2(math.e)
_QSCALE = _LOG2E / math.sqrt(_DH)
_G = 8


def _critic_body(act_ref, obs_ref, wd_ref, wlast_ref, bin_ref, wqkv_ref,
                 bqkv_ref, wo_ref, bo_ref, wout_ref, bout_ref, wv_ref,
                 bv_ref, out_ref):
    b = pl.program_id(0)
    lane = jax.lax.broadcasted_iota(jnp.int32, (1, _N), 1)

    for g in range(_G):
        a = act_ref[b * _G + g]
        kmask = lane < a
        admask = jnp.where(kmask, 0.0, _NEG)

        h = jnp.dot(obs_ref[g], wd_ref[...],
                    preferred_element_type=jnp.float32)
        h = h + (a.astype(jnp.float32) * (1.0 / _N) * wlast_ref[...]
                 + bin_ref[...])
        h = jnp.maximum(h, 0.0)

        qkv = jnp.dot(h, wqkv_ref[...],
                      preferred_element_type=jnp.float32) + bqkv_ref[...]

        parts = []
        for i in range(_H):
            qh = qkv[:, i * _DH:(i + 1) * _DH]
            kh = qkv[:, _E + i * _DH:_E + (i + 1) * _DH]
            vh = qkv[:, 2 * _E + i * _DH:2 * _E + (i + 1) * _DH]
            s = jax.lax.dot_general(qh, kh, (((1,), (1,)), ((), ())),
                                    preferred_element_type=jnp.float32)
            s = s + admask
            m = jnp.max(s, axis=1, keepdims=True)
            e = jnp.exp2(s - m)
            r = 1.0 / jnp.sum(e, axis=1, keepdims=True)
            ctx_h = jnp.dot(e, vh, preferred_element_type=jnp.float32)
            parts.append(ctx_h * r)
        ctx = jnp.concatenate(parts, axis=1)

        attn = jnp.dot(ctx, wo_ref[...],
                       preferred_element_type=jnp.float32) + bo_ref[...]
        rsa = jnp.dot(attn + h, wout_ref[...],
                      preferred_element_type=jnp.float32) + bout_ref[...]
        rsa = jnp.maximum(rsa, 0.0)

        vrow = jax.lax.dot_general(wv_ref[...], rsa, (((1,), (1,)), ((), ())),
                                   preferred_element_type=jnp.float32)
        vrow = vrow + bv_ref[...]
        vrow = jnp.where(vrow >= 0, vrow, 0.01 * vrow)
        vrow = jnp.where(kmask, vrow, 0.0)
        out_ref[g] = jnp.sum(vrow, axis=1, keepdims=True)


def _fixed(b, *_):
    return (0, 0)


def kernel(encoded_obs, actives, W_in, b_in, W_qkv, b_qkv, W_o, b_o,
           W_out, b_out, W_v, b_v):
    acts = actives.reshape(_B).astype(jnp.int32)
    qkv_scale = jnp.concatenate(
        [jnp.full((_E,), _QSCALE, jnp.float32),
         jnp.ones((2 * _E,), jnp.float32)])
    wqkv_t = W_qkv.T * qkv_scale[None, :]
    bqkv_row = (b_qkv * qkv_scale).reshape(1, 3 * _E)
    grid_spec = pltpu.PrefetchScalarGridSpec(
        num_scalar_prefetch=1,
        grid=(_B // _G,),
        in_specs=[
            pl.BlockSpec((_G, _N, _D), lambda b, *_: (b, 0, 0)),
            pl.BlockSpec((_D, _E), _fixed),
            pl.BlockSpec((1, _E), _fixed),
            pl.BlockSpec((1, _E), _fixed),
            pl.BlockSpec((_E, 3 * _E), _fixed),
            pl.BlockSpec((1, 3 * _E), _fixed),
            pl.BlockSpec((_E, _E), _fixed),
            pl.BlockSpec((1, _E), _fixed),
            pl.BlockSpec((_E, _E), _fixed),
            pl.BlockSpec((1, _E), _fixed),
            pl.BlockSpec((1, _E), _fixed),
            pl.BlockSpec((1, 1), _fixed),
        ],
        out_specs=pl.BlockSpec((_G, 1, 1), lambda b, *_: (b, 0, 0)),
    )
    out = pl.pallas_call(
        _critic_body,
        grid_spec=grid_spec,
        out_shape=jax.ShapeDtypeStruct((_B, 1, 1), jnp.float32),
        compiler_params=pltpu.CompilerParams(
            dimension_semantics=("parallel",)),
        name="critic_fused",
    )(acts, encoded_obs, W_in[:, :_D].T, W_in[:, _D].reshape(1, _E),
      b_in.reshape(1, _E), wqkv_t, bqkv_row, W_o.T, b_o.reshape(1, _E),
      W_out.T, b_out.reshape(1, _E), W_v, b_v.reshape(1, 1))
    return out.reshape(_B, 1)


# speedup vs baseline: 2.2901x; 1.0428x over previous
import math

import jax
import jax.numpy as jnp
from jax.experimental import pallas as pl
from jax.experimental.pallas import tpu as pltpu

_B, _N, _D, _E, _H = 64, 256, 256, 256, 8
_DH = _E // _H
_NEG = -1e9
_LOG2E = math.log2(math.e)
_QSCALE = _LOG2E / math.sqrt(_DH)
_G = 16


def _critic_body(act_ref, obs_ref, wd_ref, wlast_ref, bin_ref, wqkv_ref,
                 bqkv_ref, wo_ref, bo_ref, wout_ref, bout_ref, wv_ref,
                 bv_ref, out_ref):
    b = pl.program_id(0)
    lane = jax.lax.broadcasted_iota(jnp.int32, (1, _N), 1)

    for g in range(_G):
        a = act_ref[b * _G + g]
        kmask = lane < a
        admask = jnp.where(kmask, 0.0, _NEG)

        h = jnp.dot(obs_ref[g], wd_ref[...],
                    preferred_element_type=jnp.float32)
        h = h + (a.astype(jnp.float32) * (1.0 / _N) * wlast_ref[...]
                 + bin_ref[...])
        h = jnp.maximum(h, 0.0)

        qkv = jnp.dot(h, wqkv_ref[...],
                      preferred_element_type=jnp.float32) + bqkv_ref[...]

        parts = []
        for i in range(_H):
            qh = qkv[:, i * _DH:(i + 1) * _DH]
            kh = qkv[:, _E + i * _DH:_E + (i + 1) * _DH]
            vh = qkv[:, 2 * _E + i * _DH:2 * _E + (i + 1) * _DH]
            s = jax.lax.dot_general(qh, kh, (((1,), (1,)), ((), ())),
                                    preferred_element_type=jnp.float32)
            s = s + admask
            m = jnp.max(s, axis=1, keepdims=True)
            e = jnp.exp2(s - m)
            r = 1.0 / jnp.sum(e, axis=1, keepdims=True)
            ctx_h = jnp.dot(e, vh, preferred_element_type=jnp.float32)
            parts.append(ctx_h * r)
        ctx = jnp.concatenate(parts, axis=1)

        attn = jnp.dot(ctx, wo_ref[...],
                       preferred_element_type=jnp.float32) + bo_ref[...]
        rsa = jnp.dot(attn + h, wout_ref[...],
                      preferred_element_type=jnp.float32) + bout_ref[...]
        rsa = jnp.maximum(rsa, 0.0)

        vrow = jax.lax.dot_general(wv_ref[...], rsa, (((1,), (1,)), ((), ())),
                                   preferred_element_type=jnp.float32)
        vrow = vrow + bv_ref[...]
        vrow = jnp.where(vrow >= 0, vrow, 0.01 * vrow)
        vrow = jnp.where(kmask, vrow, 0.0)
        out_ref[g] = jnp.sum(vrow, axis=1, keepdims=True)


def _fixed(b, *_):
    return (0, 0)


def kernel(encoded_obs, actives, W_in, b_in, W_qkv, b_qkv, W_o, b_o,
           W_out, b_out, W_v, b_v):
    acts = actives.reshape(_B).astype(jnp.int32)
    qkv_scale = jnp.concatenate(
        [jnp.full((_E,), _QSCALE, jnp.float32),
         jnp.ones((2 * _E,), jnp.float32)])
    wqkv_t = W_qkv.T * qkv_scale[None, :]
    bqkv_row = (b_qkv * qkv_scale).reshape(1, 3 * _E)
    grid_spec = pltpu.PrefetchScalarGridSpec(
        num_scalar_prefetch=1,
        grid=(_B // _G,),
        in_specs=[
            pl.BlockSpec((_G, _N, _D), lambda b, *_: (b, 0, 0)),
            pl.BlockSpec((_D, _E), _fixed),
            pl.BlockSpec((1, _E), _fixed),
            pl.BlockSpec((1, _E), _fixed),
            pl.BlockSpec((_E, 3 * _E), _fixed),
            pl.BlockSpec((1, 3 * _E), _fixed),
            pl.BlockSpec((_E, _E), _fixed),
            pl.BlockSpec((1, _E), _fixed),
            pl.BlockSpec((_E, _E), _fixed),
            pl.BlockSpec((1, _E), _fixed),
            pl.BlockSpec((1, _E), _fixed),
            pl.BlockSpec((1, 1), _fixed),
        ],
        out_specs=pl.BlockSpec((_G, 1, 1), lambda b, *_: (b, 0, 0)),
    )
    out = pl.pallas_call(
        _critic_body,
        grid_spec=grid_spec,
        out_shape=jax.ShapeDtypeStruct((_B, 1, 1), jnp.float32),
        compiler_params=pltpu.CompilerParams(
            dimension_semantics=("parallel",)),
        name="critic_fused",
    )(acts, encoded_obs, W_in[:, :_D].T, W_in[:, _D].reshape(1, _E),
      b_in.reshape(1, _E), wqkv_t, bqkv_row, W_o.T, b_o.reshape(1, _E),
      W_out.T, b_out.reshape(1, _E), W_v, b_v.reshape(1, 1))
    return out.reshape(_B, 1)
